# baseline (speedup 1.0000x reference)
import functools

import jax
import jax.numpy as jnp
from jax import lax
from jax.experimental import pallas as pl
from jax.experimental.pallas import tpu as pltpu
from jax.experimental.pallas import tpu_sc as plsc

_IDX_PAD = 304
_NEG_INF = float("-inf")


def _topk_body(conf_ref, mask_ref, topk_ref, idxg_ref, x_ref):
    bs, N, C = conf_ref.shape
    K = N - 600

    x_ref[...] = jnp.max(conf_ref[...], axis=-1)
    ii = lax.broadcasted_iota(jnp.int32, (bs, N), 1)
    ck = lax.broadcasted_iota(jnp.int32, (bs, _IDX_PAD), 1)

    def round_fn(r, _):
        x = x_ref[...]
        mx = jnp.max(x, axis=1, keepdims=True)
        eq = x == mx
        idxc = jnp.min(jnp.where(eq, ii, N), axis=1, keepdims=True)
        sel = ck == r
        topk_ref[...] = jnp.where(sel, mx, topk_ref[...])
        idxg_ref[...] = jnp.where(sel, idxc, idxg_ref[...])
        x_ref[...] = jnp.where(ii == idxc, _NEG_INF, x)
        return 0

    lax.fori_loop(0, K, round_fn, 0, unroll=False)

    m = mask_ref[...] != 0
    raw = idxg_ref[...]
    T = N - K
    blended = jnp.where(m, raw, T + ck)
    boff = lax.broadcasted_iota(jnp.int32, (bs, _IDX_PAD), 0) * N
    idxg_ref[...] = jnp.where(ck < K, blended, ck - K) + boff


def _run_topk(confidence, mask_col):
    bs, N, C = confidence.shape
    return pl.pallas_call(
        _topk_body,
        in_specs=[
            pl.BlockSpec((bs, N, C), lambda: (0, 0, 0)),
            pl.BlockSpec((bs, 1), lambda: (0, 0)),
        ],
        out_specs=(
            pl.BlockSpec((bs, _IDX_PAD), lambda: (0, 0)),
            pl.BlockSpec((bs, _IDX_PAD), lambda: (0, 0)),
        ),
        out_shape=(
            jax.ShapeDtypeStruct((bs, _IDX_PAD), jnp.float32),
            jax.ShapeDtypeStruct((bs, _IDX_PAD), jnp.int32),
        ),
        scratch_shapes=[pltpu.VMEM((bs, N), jnp.float32)],
    )(confidence, mask_col)


def _make_sc_assemble(bs, N, T, D, A):
    K = N - T
    info = plsc.get_sparse_core_info()
    NC, NS = info.num_cores, info.num_subcores
    NW = NC * NS
    per_w = bs // NW
    mesh = plsc.VectorSubcoreMesh(core_axis_name="c", subcore_axis_name="s")

    @functools.partial(
        pl.kernel,
        mesh=mesh,
        out_type=jax.ShapeDtypeStruct((bs, N, D), jnp.float32),
        scratch_types=[
            pltpu.VMEM((_IDX_PAD,), jnp.int32),
            pltpu.VMEM((256, D), jnp.float32),
            pltpu.VMEM((48, D), jnp.float32),
            pltpu.VMEM((4, D), jnp.float32),
            pltpu.VMEM((16,), jnp.int32),
            pltpu.SemaphoreType.DMA,
            pltpu.SemaphoreType.DMA,
        ],
    )
    def sc_assemble(feat3, feat_flat, cfeat, mask_e, idxg, outf,
                    idx_v, buf_a, buf_e, buf_d, mask_v, sem_g, sem_c):
        wid = lax.axis_index("s") * NC + lax.axis_index("c")
        for j in range(per_w):
            b = wid * per_w + j
            pltpu.sync_copy(mask_e.at[b], mask_v)
            pltpu.sync_copy(idxg.at[b], idx_v)
            m = mask_v[...][0] != 0

            @pl.when(m)
            def _():
                pltpu.async_copy(cfeat.at[b], outf.at[b, pl.ds(0, T)], sem_c)

            @pl.when(jnp.logical_not(m))
            def _():
                pltpu.async_copy(feat3.at[b, pl.ds(0, T)],
                                 outf.at[b, pl.ds(0, T)], sem_c)

            cps = [
                pltpu.async_copy(feat_flat.at[idx_v.at[pl.ds(0, 128)]],
                                 buf_a.at[pl.ds(0, 128)], sem_g),
                pltpu.async_copy(feat_flat.at[idx_v.at[pl.ds(128, 128)]],
                                 buf_a.at[pl.ds(128, 128)], sem_g),
                pltpu.async_copy(feat_flat.at[idx_v.at[pl.ds(256, 48)]],
                                 buf_e, sem_g),
            ]
            for c in cps:
                c.wait()
            out_cps = [
                pltpu.async_copy(buf_a, outf.at[b, pl.ds(T, 256)], sem_g),
                pltpu.async_copy(buf_e.at[pl.ds(0, 40)],
                                 outf.at[b, pl.ds(T + 256, 40)], sem_g),
            ]
            for r in range(4):
                for c16 in range(D // 16):
                    buf_d[r, pl.ds(c16 * 16, 16)] = (
                        buf_e[40 + r, pl.ds(c16 * 16, 16)])
            out_cps.append(
                pltpu.async_copy(buf_d, outf.at[b, pl.ds(T + 296, 4)], sem_g))
            for c in out_cps:
                c.wait()
            pltpu.make_async_copy(
                cfeat.at[b], outf.at[b, pl.ds(0, T)], sem_c).wait()

    return sc_assemble


def kernel(confidence, instance_feature, anchor, cached_feature,
           cached_anchor, mask):
    bs, N, C = confidence.shape
    D = instance_feature.shape[2]
    A = anchor.shape[2]
    T = cached_feature.shape[1]
    K = N - T

    mask_i32 = mask.astype(jnp.int32)
    topk_p, idxg = _run_topk(confidence, mask_i32[:, None])
    topk = topk_p[:, :K]

    feat_flat = instance_feature.reshape(bs * N, D)
    mask_e = jnp.broadcast_to(mask_i32[:, None], (bs, 16))
    sc_assemble = _make_sc_assemble(bs, N, T, D, A)
    outf = sc_assemble(instance_feature, feat_flat, cached_feature,
                       mask_e, idxg)

    idx_loc = idxg[:, :K] - jnp.arange(bs, dtype=jnp.int32)[:, None] * N
    sel_anc = jnp.take_along_axis(anchor, idx_loc[:, :, None], axis=1)
    sela = jnp.concatenate([cached_anchor, sel_anc], axis=1)
    outa = jnp.where(mask[:, None, None], sela, anchor)
    return outf, outa, topk
